# baseline (speedup 1.0000x reference)
import functools

import jax
import jax.numpy as jnp
from jax import lax
from jax.experimental import pallas as pl
from jax.experimental.pallas import tpu as pltpu
from jax.experimental.pallas import tpu_sc as plsc

_CHUNK = 128


@functools.lru_cache(maxsize=None)
def _build(batch, n_rows, d):
    info = plsc.get_sparse_core_info()
    nw = info.num_cores * info.num_subcores
    b_per_w = batch // nw
    n_chunks = b_per_w // _CHUNK
    mesh = plsc.VectorSubcoreMesh(core_axis_name="c", subcore_axis_name="s")

    @functools.partial(
        pl.kernel,
        mesh=mesh,
        out_type=jax.ShapeDtypeStruct((batch, d), jnp.float32),
        scratch_types=[
            pltpu.VMEM((n_chunks, _CHUNK), jnp.int32),
            pltpu.VMEM((b_per_w, d), jnp.float32),
            pltpu.VMEM_SHARED((n_rows, d), jnp.float32),
            pltpu.SemaphoreType.DMA,
            pltpu.SemaphoreType.DMA,
            pltpu.SemaphoreType.DMA,
        ],
    )
    def emb_kernel(table_hbm, idx_hbm, out_hbm, idx_v, rows_v, table_sp,
                   gsem, osem, ssem):
        sid = lax.axis_index("s")
        wid = sid * info.num_cores + lax.axis_index("c")
        base = wid * b_per_w
        idx_cp = pltpu.async_copy(idx_hbm.at[wid], idx_v, gsem)

        @pl.when(sid == 0)
        def _():
            pltpu.async_copy(table_hbm, table_sp, ssem)

        idx_cp.wait()
        g0 = pltpu.async_copy(
            table_hbm.at[idx_v.at[0]], rows_v.at[pl.ds(0, _CHUNK)], gsem
        )
        g0.wait()
        outs = [
            pltpu.async_copy(
                rows_v.at[pl.ds(0, _CHUNK)], out_hbm.at[pl.ds(base, _CHUNK)],
                osem,
            )
        ]

        @pl.when(sid == 0)
        def _():
            pltpu.make_async_copy(table_hbm, table_sp, ssem).wait()

        plsc.subcore_barrier()
        gathers = [
            pltpu.async_copy(
                table_sp.at[idx_v.at[j]],
                rows_v.at[pl.ds(j * _CHUNK, _CHUNK)],
                gsem,
            )
            for j in range(1, n_chunks)
        ]
        for j in range(1, n_chunks):
            gathers[j - 1].wait()
            outs.append(
                pltpu.async_copy(
                    rows_v.at[pl.ds(j * _CHUNK, _CHUNK)],
                    out_hbm.at[pl.ds(base + j * _CHUNK, _CHUNK)],
                    osem,
                )
            )
        for c in outs:
            c.wait()

    return emb_kernel, nw, n_chunks


def kernel(labels, training, embedding_table):
    del training
    batch, = labels.shape
    n_rows, d = embedding_table.shape
    emb_kernel, nw, n_chunks = _build(batch, n_rows, d)
    idx = labels.astype(jnp.int32).reshape(nw, n_chunks, _CHUNK)
    return emb_kernel(embedding_table, idx)


# speedup vs baseline: 1.0385x; 1.0385x over previous
import functools

import jax
import jax.numpy as jnp
from jax import lax
from jax.experimental import pallas as pl
from jax.experimental.pallas import tpu as pltpu
from jax.experimental.pallas import tpu_sc as plsc

_CHUNK = 128


@functools.lru_cache(maxsize=None)
def _build(batch, n_rows, d):
    info = plsc.get_sparse_core_info()
    nw = info.num_cores * info.num_subcores
    b_per_w = batch // nw
    n_chunks = b_per_w // _CHUNK
    mesh = plsc.VectorSubcoreMesh(core_axis_name="c", subcore_axis_name="s")

    @functools.partial(
        pl.kernel,
        mesh=mesh,
        out_type=jax.ShapeDtypeStruct((batch, d), jnp.float32),
        scratch_types=[
            pltpu.VMEM((n_chunks, _CHUNK), jnp.int32),
            pltpu.VMEM((b_per_w, d), jnp.float32),
            pltpu.VMEM_SHARED((n_rows, d), jnp.float32),
            pltpu.SemaphoreType.DMA,
            pltpu.SemaphoreType.DMA,
        ],
    )
    def emb_kernel(table_hbm, idx_hbm, out_hbm, idx_v, rows_v, table_sp,
                   gsem, osem):
        sid = lax.axis_index("s")
        wid = sid * info.num_cores + lax.axis_index("c")
        base = wid * b_per_w
        idx_cp = pltpu.async_copy(idx_hbm.at[wid], idx_v, gsem)

        @pl.when(sid == 0)
        def _():
            pltpu.sync_copy(table_hbm, table_sp)

        plsc.subcore_barrier()
        idx_cp.wait()

        def issue(j, carry):
            pltpu.async_copy(
                table_sp.at[idx_v.at[j]],
                rows_v.at[pl.ds(j * _CHUNK, _CHUNK)],
                gsem,
            )
            return carry

        lax.fori_loop(0, n_chunks, issue, 0)

        def drain(j, carry):
            pltpu.make_async_copy(
                table_sp.at[idx_v.at[j]],
                rows_v.at[pl.ds(j * _CHUNK, _CHUNK)],
                gsem,
            ).wait()
            pltpu.async_copy(
                rows_v.at[pl.ds(j * _CHUNK, _CHUNK)],
                out_hbm.at[pl.ds(base + j * _CHUNK, _CHUNK)],
                osem,
            )
            return carry

        lax.fori_loop(0, n_chunks, drain, 0)
        pltpu.make_async_copy(
            rows_v, out_hbm.at[pl.ds(base, b_per_w)], osem
        ).wait()

    return emb_kernel, nw, n_chunks


def kernel(labels, training, embedding_table):
    del training
    batch, = labels.shape
    n_rows, d = embedding_table.shape
    emb_kernel, nw, n_chunks = _build(batch, n_rows, d)
    idx = labels.astype(jnp.int32).reshape(nw, n_chunks, _CHUNK)
    return emb_kernel(embedding_table, idx)
